# Initial kernel scaffold; baseline (speedup 1.0000x reference)
import functools

import jax
import jax.numpy as jnp
from jax import lax
from jax.experimental import pallas as pl
from jax.experimental.pallas import tpu as pltpu
from jax.experimental.pallas import tpu_sc as plsc

N = 10000
E = 320000
IN_DIM = 134
HID = 256

NC = 2
NS = 16
CH = 128
EPT = 10240
E_PAD = NC * NS * EPT
NPAD = 10240
D2 = 144
ROWS_PT = N // NS

BM = 1000


def _deg_body(dst_hbm, w_hbm, out_hbm, deg_sh, dst_v, w_v, zb):
    cid = lax.axis_index("c")
    sid = lax.axis_index("s")
    z16 = jnp.zeros((16,), jnp.float32)

    def zfill(i, c):
        zb[pl.ds(i * 16, 16)] = z16
        return c
    lax.fori_loop(0, 640 // 16, zfill, 0)
    pltpu.sync_copy(zb, deg_sh.at[pl.ds(sid * 640, 640)])
    plsc.subcore_barrier()

    base = (cid * NS + sid) * EPT

    def chunk(i, c):
        off = base + i * CH
        pltpu.sync_copy(dst_hbm.at[pl.ds(off, CH)], dst_v)
        pltpu.sync_copy(w_hbm.at[pl.ds(off, CH)], w_v)
        pltpu.sync_copy(w_v, deg_sh.at[dst_v], add=True)
        return c
    lax.fori_loop(0, EPT // CH, chunk, 0)
    plsc.subcore_barrier()
    pltpu.sync_copy(deg_sh.at[pl.ds(sid * 640, 640)],
                    out_hbm.at[pl.ds(cid * NPAD + sid * 640, 640)])


def _make_agg_body(d, feature_split):
    nsl = d // 16
    chunks = (2 * EPT if feature_split else EPT) // CH

    def body(g_hbm, src_hbm, dst_hbm, w_hbm, out_hbm,
             acc_sh, src_v, dst_v, w_v, rows_v, zb, sem):
        cid = lax.axis_index("c")
        sid = lax.axis_index("s")
        z16 = jnp.zeros((16,), jnp.float32)

        def zfill(i, c):
            r = i // nsl
            j = i - r * nsl
            zb[r, pl.ds(j * 16, 16)] = z16
            return c
        lax.fori_loop(0, 25 * nsl, zfill, 0)

        def zcp(i, c):
            pltpu.sync_copy(zb, acc_sh.at[pl.ds(sid * ROWS_PT + i * 25, 25)])
            return c
        lax.fori_loop(0, ROWS_PT // 25, zcp, 0)
        plsc.subcore_barrier()

        if feature_split:
            ebase = sid * (2 * EPT)
            sbase = cid * E_PAD + ebase
        else:
            ebase = (cid * NS + sid) * EPT
            sbase = ebase

        def chunk(i, c):
            off = ebase + i * CH
            soff = sbase + i * CH
            pltpu.sync_copy(src_hbm.at[pl.ds(soff, CH)], src_v)
            pltpu.sync_copy(dst_hbm.at[pl.ds(off, CH)], dst_v)
            pltpu.sync_copy(w_hbm.at[pl.ds(off, CH)], w_v)
            pltpu.async_copy(g_hbm.at[src_v], rows_v, sem).wait()

            def edge(e, c2):
                ws = w_v[e]
                for j in range(nsl):
                    sl = pl.ds(j * 16, 16)
                    rows_v[e, sl] = rows_v[e, sl] * ws
                return c2
            lax.fori_loop(0, CH, edge, 0)
            pltpu.sync_copy(rows_v, acc_sh.at[dst_v], add=True)
            return c
        lax.fori_loop(0, chunks, chunk, 0)
        plsc.subcore_barrier()
        pltpu.sync_copy(acc_sh.at[pl.ds(sid * ROWS_PT, ROWS_PT)],
                        out_hbm.at[pl.ds(cid * N + sid * ROWS_PT, ROWS_PT)])

    return body


def _make_agg_call(d, feature_split):
    return pl.kernel(
        _make_agg_body(d, feature_split),
        out_type=jax.ShapeDtypeStruct((2 * N, d), jnp.float32),
        mesh=plsc.VectorSubcoreMesh(core_axis_name="c", subcore_axis_name="s"),
        scratch_types=[
            pltpu.VMEM_SHARED((N, d), jnp.float32),
            pltpu.VMEM((CH,), jnp.int32),
            pltpu.VMEM((CH,), jnp.int32),
            pltpu.VMEM((CH,), jnp.float32),
            pltpu.VMEM((CH, d), jnp.float32),
            pltpu.VMEM((25, d), jnp.float32),
            pltpu.SemaphoreType.DMA,
        ],
    )


def _tc1_body(x_ref, w_ref, d0_ref, d1_ref, dis_ref, g_ref):
    deg = d0_ref[...] + d1_ref[...] + 1.0
    dis = lax.rsqrt(deg)
    dis_ref[...] = dis
    h = lax.dot_general(x_ref[...], w_ref[...], (((1,), (0,)), ((), ())),
                        precision=lax.Precision.HIGHEST,
                        preferred_element_type=jnp.float32)
    g = h * dis
    g_ref[0, :, :] = g[:, :128]
    g_ref[1, :, :] = g[:, 128:]


def _tc2_body(a_ref, g1_ref, dis_ref, b1_ref, w2_ref, g2_ref):
    dis = dis_ref[...]
    zlo = jnp.maximum(dis * (a_ref[0] + g1_ref[0]) + b1_ref[0], 0.0)
    zhi = jnp.maximum(dis * (a_ref[1] + g1_ref[1]) + b1_ref[1], 0.0)
    dn = (((1,), (0,)), ((), ()))
    h2 = (lax.dot_general(zlo, w2_ref[0:128, :], dn,
                          precision=lax.Precision.HIGHEST,
                          preferred_element_type=jnp.float32)
          + lax.dot_general(zhi, w2_ref[128:256, :], dn,
                            precision=lax.Precision.HIGHEST,
                            preferred_element_type=jnp.float32))
    g2_ref[...] = h2 * dis


def _tc3_body(a_ref, g2_ref, dis_ref, b2_ref, o_ref):
    dis = dis_ref[...]
    val = dis * (a_ref[0] + a_ref[1] + g2_ref[...]) + b2_ref[...]
    o_ref[...] = val[:, :IN_DIM]


def kernel(x, edge_index, edge_weight, W1, b1, W2, b2):
    src = edge_index[0]
    dst = edge_index[1]
    npad = E_PAD - E
    padi = (jnp.arange(npad, dtype=jnp.int32) % N)
    src_p = jnp.concatenate([src, padi])
    dst_p = jnp.concatenate([dst, padi])
    w_p = jnp.concatenate([edge_weight, jnp.zeros((npad,), jnp.float32)])
    src2_p = jnp.concatenate([src_p, src_p + N])

    x_pad = jnp.pad(x, ((0, 0), (0, HID - IN_DIM)))
    W1_pad = jnp.pad(W1, ((0, HID - IN_DIM), (0, 0)))
    W2_pad = jnp.pad(W2, ((0, 0), (0, D2 - IN_DIM)))
    b1s = b1.reshape(2, 1, 128)
    b2p = jnp.pad(b2, (0, D2 - IN_DIM)).reshape(1, D2)

    deg_call = pl.kernel(
        _deg_body,
        out_type=jax.ShapeDtypeStruct((2 * NPAD,), jnp.float32),
        mesh=plsc.VectorSubcoreMesh(core_axis_name="c", subcore_axis_name="s"),
        scratch_types=[
            pltpu.VMEM_SHARED((NPAD,), jnp.float32),
            pltpu.VMEM((CH,), jnp.int32),
            pltpu.VMEM((CH,), jnp.float32),
            pltpu.VMEM((640,), jnp.float32),
        ],
    )
    degp = deg_call(dst_p, w_p)
    d0 = degp[:N].reshape(N, 1)
    d1 = degp[NPAD:NPAD + N].reshape(N, 1)

    tc1 = pl.pallas_call(
        _tc1_body,
        grid=(N // BM,),
        in_specs=[
            pl.BlockSpec((BM, HID), lambda m: (m, 0)),
            pl.BlockSpec((HID, HID), lambda m: (0, 0)),
            pl.BlockSpec((BM, 1), lambda m: (m, 0)),
            pl.BlockSpec((BM, 1), lambda m: (m, 0)),
        ],
        out_specs=[
            pl.BlockSpec((BM, 1), lambda m: (m, 0)),
            pl.BlockSpec((2, BM, 128), lambda m: (0, m, 0)),
        ],
        out_shape=[
            jax.ShapeDtypeStruct((N, 1), jnp.float32),
            jax.ShapeDtypeStruct((2, N, 128), jnp.float32),
        ],
    )
    dis, g1 = tc1(x_pad, W1_pad, d0, d1)

    agg1_call = _make_agg_call(128, feature_split=True)
    agg1 = agg1_call(g1.reshape(2 * N, 128), src2_p, dst_p, w_p)

    tc2 = pl.pallas_call(
        _tc2_body,
        grid=(N // BM,),
        in_specs=[
            pl.BlockSpec((2, BM, 128), lambda m: (0, m, 0)),
            pl.BlockSpec((2, BM, 128), lambda m: (0, m, 0)),
            pl.BlockSpec((BM, 1), lambda m: (m, 0)),
            pl.BlockSpec((2, 1, 128), lambda m: (0, 0, 0)),
            pl.BlockSpec((HID, D2), lambda m: (0, 0)),
        ],
        out_specs=pl.BlockSpec((BM, D2), lambda m: (m, 0)),
        out_shape=jax.ShapeDtypeStruct((N, D2), jnp.float32),
    )
    g2 = tc2(agg1.reshape(2, N, 128), g1, dis, b1s, W2_pad)

    agg2_call = _make_agg_call(D2, feature_split=False)
    agg2 = agg2_call(g2, src_p, dst_p, w_p)

    tc3 = pl.pallas_call(
        _tc3_body,
        grid=(N // BM,),
        in_specs=[
            pl.BlockSpec((2, BM, D2), lambda m: (0, m, 0)),
            pl.BlockSpec((BM, D2), lambda m: (m, 0)),
            pl.BlockSpec((BM, 1), lambda m: (m, 0)),
            pl.BlockSpec((1, D2), lambda m: (0, 0)),
        ],
        out_specs=pl.BlockSpec((BM, IN_DIM), lambda m: (m, 0)),
        out_shape=jax.ShapeDtypeStruct((N, IN_DIM), jnp.float32),
    )
    out = tc3(agg2.reshape(2, N, D2), g2, dis, b2p)
    return out


# speedup vs baseline: 8.7690x; 8.7690x over previous
import functools

import jax
import jax.numpy as jnp
from jax import lax
from jax.experimental import pallas as pl
from jax.experimental.pallas import tpu as pltpu
from jax.experimental.pallas import tpu_sc as plsc

N = 10000
E = 320000
IN_DIM = 134
HID = 256

NC = 2
NS = 16
CH = 128
EPT = 10240
E_PAD = NC * NS * EPT
NPAD = 10240
D2 = 144
ROWS_PT = NPAD // NS

BM = 1024


def _deg_body(dst_hbm, w_hbm, out_hbm, deg_sh, dst_v, w_v, zb):
    cid = lax.axis_index("c")
    sid = lax.axis_index("s")
    z16 = jnp.zeros((16,), jnp.float32)

    def zfill(i, c):
        zb[pl.ds(i * 16, 16)] = z16
        return c
    lax.fori_loop(0, 640 // 16, zfill, 0)
    pltpu.sync_copy(zb, deg_sh.at[pl.ds(sid * 640, 640)])
    plsc.subcore_barrier()

    base = (cid * NS + sid) * EPT

    def chunk(i, c):
        off = base + i * CH
        pltpu.sync_copy(dst_hbm.at[pl.ds(off, CH)], dst_v)
        pltpu.sync_copy(w_hbm.at[pl.ds(off, CH)], w_v)
        pltpu.sync_copy(w_v, deg_sh.at[dst_v], add=True)
        return c
    lax.fori_loop(0, EPT // CH, chunk, 0)
    plsc.subcore_barrier()
    pltpu.sync_copy(deg_sh.at[pl.ds(sid * 640, 640)],
                    out_hbm.at[pl.ds(cid * NPAD + sid * 640, 640)])


def _make_agg_body(d, feature_split):
    nsl = d // 16
    chunks = (2 * EPT if feature_split else EPT) // CH

    def body(g_hbm, src_hbm, dst_hbm, w_hbm, out_hbm,
             acc_sh, src_v, dst_v, w_v, rows_v, zb, sem):
        cid = lax.axis_index("c")
        sid = lax.axis_index("s")
        z16 = jnp.zeros((16,), jnp.float32)

        def zfill(i, c):
            r = i // nsl
            j = i - r * nsl
            zb[r, pl.ds(j * 16, 16)] = z16
            return c
        lax.fori_loop(0, 32 * nsl, zfill, 0)

        def zcp(i, c):
            pltpu.sync_copy(zb, acc_sh.at[pl.ds(sid * ROWS_PT + i * 32, 32)])
            return c
        lax.fori_loop(0, ROWS_PT // 32, zcp, 0)
        plsc.subcore_barrier()

        if feature_split:
            ebase = sid * (2 * EPT)
            sbase = cid * E_PAD + ebase
        else:
            ebase = (cid * NS + sid) * EPT
            sbase = ebase

        def chunk(i, c):
            off = ebase + i * CH
            soff = sbase + i * CH
            pltpu.sync_copy(src_hbm.at[pl.ds(soff, CH)], src_v)
            pltpu.sync_copy(dst_hbm.at[pl.ds(off, CH)], dst_v)
            pltpu.sync_copy(w_hbm.at[pl.ds(off, CH)], w_v)
            pltpu.async_copy(g_hbm.at[src_v], rows_v, sem).wait()

            def grp(g, c2):
                wvec = w_v[pl.ds(g * 16, 16)]
                for k in range(16):
                    ws = wvec[k]
                    e = g * 16 + k
                    for j in range(nsl):
                        sl = pl.ds(j * 16, 16)
                        rows_v[e, sl] = rows_v[e, sl] * ws
                return c2
            lax.fori_loop(0, CH // 16, grp, 0)
            pltpu.sync_copy(rows_v, acc_sh.at[dst_v], add=True)
            return c
        lax.fori_loop(0, chunks, chunk, 0)
        plsc.subcore_barrier()
        pltpu.sync_copy(acc_sh.at[pl.ds(sid * ROWS_PT, ROWS_PT)],
                        out_hbm.at[pl.ds(cid * NPAD + sid * ROWS_PT, ROWS_PT)])

    return body


def _make_agg_call(d, feature_split):
    params = (None if d % 128 == 0
              else pltpu.CompilerParams(use_tc_tiling_on_sc=False))
    return pl.kernel(
        _make_agg_body(d, feature_split),
        out_type=jax.ShapeDtypeStruct((2 * NPAD, d), jnp.float32),
        mesh=plsc.VectorSubcoreMesh(core_axis_name="c", subcore_axis_name="s"),
        compiler_params=params,
        scratch_types=[
            pltpu.VMEM_SHARED((NPAD, d), jnp.float32),
            pltpu.VMEM((CH,), jnp.int32),
            pltpu.VMEM((CH,), jnp.int32),
            pltpu.VMEM((CH,), jnp.float32),
            pltpu.VMEM((CH, d), jnp.float32),
            pltpu.VMEM((32, d), jnp.float32),
            pltpu.SemaphoreType.DMA,
        ],
    )


def _tc1_body(x_ref, w_ref, d0_ref, d1_ref, dis_ref, g_ref):
    deg = d0_ref[...] + d1_ref[...] + 1.0
    dis = lax.rsqrt(deg)
    dis_ref[...] = dis
    h = lax.dot_general(x_ref[...], w_ref[...], (((1,), (0,)), ((), ())),
                        precision=lax.Precision.HIGHEST,
                        preferred_element_type=jnp.float32)
    g = h * dis
    g_ref[0, :, :] = g[:, :128]
    g_ref[1, :, :] = g[:, 128:]


def _tc2_body(a_ref, g1_ref, dis_ref, b1_ref, w2_ref, g2_ref):
    dis = dis_ref[...]
    zlo = jnp.maximum(dis * (a_ref[0] + g1_ref[0]) + b1_ref[0], 0.0)
    zhi = jnp.maximum(dis * (a_ref[1] + g1_ref[1]) + b1_ref[1], 0.0)
    dn = (((1,), (0,)), ((), ()))
    h2 = (lax.dot_general(zlo, w2_ref[0:128, :], dn,
                          precision=lax.Precision.HIGHEST,
                          preferred_element_type=jnp.float32)
          + lax.dot_general(zhi, w2_ref[128:256, :], dn,
                            precision=lax.Precision.HIGHEST,
                            preferred_element_type=jnp.float32))
    g2_ref[...] = h2 * dis


def _tc3_body(a_ref, g2_ref, dis_ref, b2_ref, o_ref):
    dis = dis_ref[...]
    val = dis * (a_ref[0] + a_ref[1] + g2_ref[...]) + b2_ref[...]
    o_ref[...] = val[:, :IN_DIM]


def kernel(x, edge_index, edge_weight, W1, b1, W2, b2):
    src = edge_index[0]
    dst = edge_index[1]
    npad = E_PAD - E
    padi = (jnp.arange(npad, dtype=jnp.int32) % N)
    src_p = jnp.concatenate([src, padi])
    dst_p = jnp.concatenate([dst, padi])
    w_p = jnp.concatenate([edge_weight, jnp.zeros((npad,), jnp.float32)])
    src2_p = jnp.concatenate([src_p, src_p + NPAD])

    x_pad = jnp.pad(x, ((0, NPAD - N), (0, HID - IN_DIM)))
    W1_pad = jnp.pad(W1, ((0, HID - IN_DIM), (0, 0)))
    W2_pad = jnp.pad(W2, ((0, 0), (0, D2 - IN_DIM)))
    b1s = b1.reshape(2, 1, 128)
    b2p = jnp.pad(b2, (0, D2 - IN_DIM)).reshape(1, D2)

    deg_call = pl.kernel(
        _deg_body,
        out_type=jax.ShapeDtypeStruct((2 * NPAD,), jnp.float32),
        mesh=plsc.VectorSubcoreMesh(core_axis_name="c", subcore_axis_name="s"),
        scratch_types=[
            pltpu.VMEM_SHARED((NPAD,), jnp.float32),
            pltpu.VMEM((CH,), jnp.int32),
            pltpu.VMEM((CH,), jnp.float32),
            pltpu.VMEM((640,), jnp.float32),
        ],
    )
    degp = deg_call(dst_p, w_p)
    d0 = degp[:NPAD].reshape(NPAD, 1)
    d1 = degp[NPAD:].reshape(NPAD, 1)

    tc1 = pl.pallas_call(
        _tc1_body,
        grid=(NPAD // BM,),
        in_specs=[
            pl.BlockSpec((BM, HID), lambda m: (m, 0)),
            pl.BlockSpec((HID, HID), lambda m: (0, 0)),
            pl.BlockSpec((BM, 1), lambda m: (m, 0)),
            pl.BlockSpec((BM, 1), lambda m: (m, 0)),
        ],
        out_specs=[
            pl.BlockSpec((BM, 1), lambda m: (m, 0)),
            pl.BlockSpec((2, BM, 128), lambda m: (0, m, 0)),
        ],
        out_shape=[
            jax.ShapeDtypeStruct((NPAD, 1), jnp.float32),
            jax.ShapeDtypeStruct((2, NPAD, 128), jnp.float32),
        ],
    )
    dis, g1 = tc1(x_pad, W1_pad, d0, d1)

    agg1_call = _make_agg_call(128, feature_split=True)
    agg1 = agg1_call(g1.reshape(2 * NPAD, 128), src2_p, dst_p, w_p)

    tc2 = pl.pallas_call(
        _tc2_body,
        grid=(NPAD // BM,),
        in_specs=[
            pl.BlockSpec((2, BM, 128), lambda m: (0, m, 0)),
            pl.BlockSpec((2, BM, 128), lambda m: (0, m, 0)),
            pl.BlockSpec((BM, 1), lambda m: (m, 0)),
            pl.BlockSpec((2, 1, 128), lambda m: (0, 0, 0)),
            pl.BlockSpec((HID, D2), lambda m: (0, 0)),
        ],
        out_specs=pl.BlockSpec((BM, D2), lambda m: (m, 0)),
        out_shape=jax.ShapeDtypeStruct((NPAD, D2), jnp.float32),
    )
    g2 = tc2(agg1.reshape(2, NPAD, 128), g1, dis, b1s, W2_pad)

    agg2_call = _make_agg_call(D2, feature_split=False)
    agg2 = agg2_call(g2, src_p, dst_p, w_p)

    tc3 = pl.pallas_call(
        _tc3_body,
        grid=(NPAD // BM,),
        in_specs=[
            pl.BlockSpec((2, BM, D2), lambda m: (0, m, 0)),
            pl.BlockSpec((BM, D2), lambda m: (m, 0)),
            pl.BlockSpec((BM, 1), lambda m: (m, 0)),
            pl.BlockSpec((1, D2), lambda m: (0, 0)),
        ],
        out_specs=pl.BlockSpec((BM, IN_DIM), lambda m: (m, 0)),
        out_shape=jax.ShapeDtypeStruct((NPAD, IN_DIM), jnp.float32),
    )
    out = tc3(agg2.reshape(2, NPAD, D2), g2, dis, b2p)
    return out[:N]
